# baseline (speedup 1.0000x reference)
import jax
import jax.numpy as jnp
from jax import lax
from jax.experimental import pallas as pl
from jax.experimental.pallas import tpu as pltpu
from jax.experimental.pallas import tpu_sc as plsc

_L = 16
_ROWS, _N = 128, 32768
_UN = 8
_UN2 = 4
_GW = 16
_PV = 256
_CAP2 = 256
_NEWTON_ITERS = 10
_NEG = -3.0e38


def _sc_body(x_hbm, o_hbm, rb0, rb1, zbuf, gbuf, pairbuf, c2vals, c2pos,
             sem_in, sem_out):
    info = plsc.get_sparse_core_info()
    nc, ns = info.num_cores, info.num_subcores
    rpw = _ROWS // (nc * ns)
    wid = lax.axis_index("s") * nc + lax.axis_index("c")
    lane = lax.iota(jnp.int32, _L)
    zero = jnp.zeros((_L,), jnp.float32)
    izero = jnp.zeros((_L,), jnp.int32)
    sent = jnp.full((_L,), _NEG, jnp.float32)
    neg = jnp.full((_L,), _NEG, jnp.float32)
    rbufs = [rb0, rb1]
    row0 = wid * rpw
    _NG = _N // (_GW * _L)

    hin = {0: pltpu.async_copy(x_hbm.at[row0], rbufs[0], sem_in)}

    def z_body(i, _):
        for u in range(_UN):
            zbuf[pl.ds((i * _UN + u) * _L, _L)] = zero
        return 0
    lax.fori_loop(0, _N // _L // _UN, z_body, 0)

    def zp_body(i, _):
        pairbuf[pl.ds(i * _L, _L)] = izero
        return 0
    lax.fori_loop(0, _PV, zp_body, 0)

    hout = None
    prev_n = None

    for r in range(rpw):
        row = row0 + r
        hin[r].wait()
        if r + 1 < rpw:
            hin[r + 1] = pltpu.async_copy(
                x_hbm.at[row + 1], rbufs[(r + 1) % 2], sem_in)
        rbuf = rbufs[r % 2]

        def max_body(gi, acc):
            for u in range(2):
                g = gi * 2 + u
                t = [rbuf[pl.ds((g * _GW + j) * _L, _L)] for j in range(_GW)]
                while len(t) > 1:
                    t = [jnp.maximum(t[i], t[i + 1])
                         for i in range(0, len(t), 2)]
                gbuf[pl.ds(g * _L, _L)] = t[0]
                acc = jnp.maximum(acc, t[0])
            return acc

        acc = lax.fori_loop(0, _NG // 2, max_body, neg)
        m = jnp.max(acc)
        thr = jnp.broadcast_to(m - 1.0, (_L,))

        if hout is not None:
            hout.wait()

            def rst_body(i, _):
                for u in range(_UN2):
                    p = c2pos[pl.ds((i * _UN2 + u) * _L, _L)]
                    plsc.store_scatter(zbuf, [p], zero)
                return 0
            lax.fori_loop(0, prev_n, rst_body, 0)

        def pair_body(i, cur):
            for u in range(_UN2):
                g = i * _UN2 + u
                gm = gbuf[pl.ds(g * _L, _L)]
                keep = gm > thr
                base = jnp.broadcast_to(g * (_GW * _L), (_L,)) + lane
                plsc.store_scatter(pairbuf, [cur], base, mask=keep)
                cur = cur + jnp.where(keep, _L, 0)
            return cur

        pcur = lax.fori_loop(0, _NG // _UN2, pair_body, lane)
        pcnt = lax.shift_right_logical(pcur - lane, 4)
        kmax = jnp.max(pcnt)

        def scan_body(k, c):
            cur, cnt, vs = c
            b = pairbuf[pl.ds(k * _L, _L)]
            vp = k < pcnt
            for j in range(_GW):
                idx = b + j * _L
                v = plsc.load_gather(rbuf, [idx])
                keep = vp & (v > thr)
                plsc.store_scatter(c2vals, [cur], v, mask=keep)
                plsc.store_scatter(c2pos, [cur], idx, mask=keep)
                cur = cur + jnp.where(keep, _L, 0)
                cnt = cnt + jnp.where(keep, 1, 0)
                vs = vs + jnp.where(keep, v, 0.0)
            return cur, cnt, vs

        _, cnt2, vsum = lax.fori_loop(0, kmax, scan_body,
                                      (lane, izero, zero))
        nch2 = jnp.max(cnt2)
        n_new = (nch2 + (_UN2 - 1)) // _UN2
        sv0 = jnp.broadcast_to(jnp.sum(vsum) - 1.0, (_L,))
        kv0 = jnp.broadcast_to(jnp.sum(cnt2).astype(jnp.float32), (_L,))
        tau1 = sv0 / kv0

        def fill_body(j, _):
            off = jnp.minimum(j, _CAP2 - 1) * _L
            mask = j >= cnt2
            c2vals[pl.ds(off, _L)] = jnp.where(
                mask, sent, c2vals[pl.ds(off, _L)])
            c2pos[pl.ds(off, _L)] = jnp.where(
                mask, izero, c2pos[pl.ds(off, _L)])
            return 0
        lax.fori_loop(0, n_new * _UN2, fill_body, 0)

        def newton(_, t):
            def ch(i, acc2):
                sacc, kacc = acc2
                for u in range(_UN2):
                    v = c2vals[pl.ds((i * _UN2 + u) * _L, _L)]
                    act = v > t
                    sacc = sacc + jnp.where(act, v, 0.0)
                    kacc = kacc + jnp.where(act, 1.0, 0.0)
                return sacc, kacc
            sacc, kacc = lax.fori_loop(0, n_new, ch, (zero, zero))
            sv = jnp.broadcast_to(jnp.sum(sacc) - 1.0, (_L,))
            kv = jnp.broadcast_to(jnp.sum(kacc), (_L,))
            return sv / kv

        tau = lax.fori_loop(0, _NEWTON_ITERS, newton, tau1)

        def sc_out(i, _):
            for u in range(_UN2):
                j = i * _UN2 + u
                v = c2vals[pl.ds(j * _L, _L)]
                p = c2pos[pl.ds(j * _L, _L)]
                plsc.store_scatter(zbuf, [p], jnp.maximum(v - tau, 0.0),
                                   mask=v > thr)
            return 0
        lax.fori_loop(0, n_new, sc_out, 0)

        hout = pltpu.async_copy(zbuf, o_hbm.at[row], sem_out)
        prev_n = n_new

    hout.wait()


@jax.jit
def kernel(input_tensor):
    mesh = plsc.VectorSubcoreMesh(core_axis_name="c", subcore_axis_name="s")
    return pl.kernel(
        _sc_body,
        out_type=jax.ShapeDtypeStruct((_ROWS, _N), jnp.float32),
        mesh=mesh,
        scratch_types=[
            pltpu.VMEM((_N,), jnp.float32),
            pltpu.VMEM((_N,), jnp.float32),
            pltpu.VMEM((_N,), jnp.float32),
            pltpu.VMEM((_N // _GW,), jnp.float32),
            pltpu.VMEM((_L * _PV,), jnp.int32),
            pltpu.VMEM((_L * _CAP2,), jnp.float32),
            pltpu.VMEM((_L * _CAP2,), jnp.int32),
            pltpu.SemaphoreType.DMA,
            pltpu.SemaphoreType.DMA,
        ],
        compiler_params=pltpu.CompilerParams(needs_layout_passes=False),
    )(input_tensor)


# speedup vs baseline: 1.0137x; 1.0137x over previous
import jax
import jax.numpy as jnp
from jax import lax
from jax.experimental import pallas as pl
from jax.experimental.pallas import tpu as pltpu
from jax.experimental.pallas import tpu_sc as plsc

_L = 16
_ROWS, _N = 128, 32768
_UN = 8
_UN2 = 4
_GW = 16
_PV = 256
_CAP2 = 256
_NEWTON_ITERS = 10
_NEG = -3.0e38


def _sc_body(x_hbm, o_hbm, rb0, rb1, zbuf, gbuf, pairbuf, c2vals, c2pos,
             sem_in, sem_out):
    info = plsc.get_sparse_core_info()
    nc, ns = info.num_cores, info.num_subcores
    rpw = _ROWS // (nc * ns)
    wid = lax.axis_index("s") * nc + lax.axis_index("c")
    lane = lax.iota(jnp.int32, _L)
    zero = jnp.zeros((_L,), jnp.float32)
    izero = jnp.zeros((_L,), jnp.int32)
    sent = jnp.full((_L,), _NEG, jnp.float32)
    neg = jnp.full((_L,), _NEG, jnp.float32)
    rbufs = [rb0, rb1]
    row0 = wid * rpw
    _NG = _N // (_GW * _L)

    hin = {0: pltpu.async_copy(x_hbm.at[row0], rbufs[0], sem_in)}

    def z_body(i, _):
        for u in range(_UN):
            zbuf[pl.ds((i * _UN + u) * _L, _L)] = zero
        return 0
    lax.fori_loop(0, _N // _L // _UN, z_body, 0)

    def zp_body(i, _):
        pairbuf[pl.ds(i * _L, _L)] = izero
        return 0
    lax.fori_loop(0, _PV, zp_body, 0)

    hout = None
    prev_n = None

    for r in range(rpw):
        row = row0 + r
        hin[r].wait()
        if r + 1 < rpw:
            hin[r + 1] = pltpu.async_copy(
                x_hbm.at[row + 1], rbufs[(r + 1) % 2], sem_in)
        rbuf = rbufs[r % 2]

        def max_body(g, acc):
            t = [rbuf[pl.ds((g * _GW + j) * _L, _L)] for j in range(_GW)]
            while len(t) > 1:
                t = [jnp.maximum(t[i], t[i + 1]) for i in range(0, len(t), 2)]
            gbuf[pl.ds(g * _L, _L)] = t[0]
            return jnp.maximum(acc, t[0])

        acc = lax.fori_loop(0, _NG, max_body, neg)
        m = jnp.max(acc)
        thr = jnp.broadcast_to(m - 1.0, (_L,))

        if hout is not None:
            hout.wait()

            def rst_body(i, _):
                for u in range(_UN2):
                    p = c2pos[pl.ds((i * _UN2 + u) * _L, _L)]
                    plsc.store_scatter(zbuf, [p], zero)
                return 0
            lax.fori_loop(0, prev_n, rst_body, 0)

        def pair_body(i, cur):
            for u in range(_UN2):
                g = i * _UN2 + u
                gm = gbuf[pl.ds(g * _L, _L)]
                keep = gm > thr
                base = jnp.broadcast_to(g * (_GW * _L), (_L,)) + lane
                plsc.store_scatter(pairbuf, [cur], base, mask=keep)
                cur = cur + jnp.where(keep, _L, 0)
            return cur

        pcur = lax.fori_loop(0, _NG // _UN2, pair_body, lane)
        pcnt = lax.shift_right_logical(pcur - lane, 4)
        kmax = jnp.max(pcnt)

        def scan_body(k, c):
            cur, cnt, vs = c
            b = pairbuf[pl.ds(k * _L, _L)]
            vp = k < pcnt
            for j in range(_GW):
                idx = b + j * _L
                v = plsc.load_gather(rbuf, [idx])
                keep = vp & (v > thr)
                plsc.store_scatter(c2vals, [cur], v, mask=keep)
                plsc.store_scatter(c2pos, [cur], idx, mask=keep)
                cur = cur + jnp.where(keep, _L, 0)
                cnt = cnt + jnp.where(keep, 1, 0)
                vs = vs + jnp.where(keep, v, 0.0)
            return cur, cnt, vs

        _, cnt2, vsum = lax.fori_loop(0, kmax, scan_body,
                                      (lane, izero, zero))
        nch2 = jnp.max(cnt2)
        n_new = (nch2 + (_UN2 - 1)) // _UN2
        sv0 = jnp.broadcast_to(jnp.sum(vsum) - 1.0, (_L,))
        kv0 = jnp.broadcast_to(jnp.sum(cnt2).astype(jnp.float32), (_L,))
        tau1 = sv0 / kv0

        def fill_body(j, _):
            off = jnp.minimum(j, _CAP2 - 1) * _L
            mask = j >= cnt2
            c2vals[pl.ds(off, _L)] = jnp.where(
                mask, sent, c2vals[pl.ds(off, _L)])
            c2pos[pl.ds(off, _L)] = jnp.where(
                mask, izero, c2pos[pl.ds(off, _L)])
            return 0
        lax.fori_loop(0, n_new * _UN2, fill_body, 0)

        def newton(_, t):
            def ch(i, acc2):
                sacc, kacc = acc2
                for u in range(_UN2):
                    v = c2vals[pl.ds((i * _UN2 + u) * _L, _L)]
                    act = v > t
                    sacc = sacc + jnp.where(act, v, 0.0)
                    kacc = kacc + jnp.where(act, 1.0, 0.0)
                return sacc, kacc
            sacc, kacc = lax.fori_loop(0, n_new, ch, (zero, zero))
            sv = jnp.broadcast_to(jnp.sum(sacc) - 1.0, (_L,))
            kv = jnp.broadcast_to(jnp.sum(kacc), (_L,))
            return sv / kv

        tau = lax.fori_loop(0, _NEWTON_ITERS, newton, tau1)

        def sc_out(i, _):
            for u in range(_UN2):
                j = i * _UN2 + u
                v = c2vals[pl.ds(j * _L, _L)]
                p = c2pos[pl.ds(j * _L, _L)]
                plsc.store_scatter(zbuf, [p], jnp.maximum(v - tau, 0.0),
                                   mask=v > thr)
            return 0
        lax.fori_loop(0, n_new, sc_out, 0)

        hout = pltpu.async_copy(zbuf, o_hbm.at[row], sem_out)
        prev_n = n_new

    hout.wait()


@jax.jit
def kernel(input_tensor):
    mesh = plsc.VectorSubcoreMesh(core_axis_name="c", subcore_axis_name="s")
    return pl.kernel(
        _sc_body,
        out_type=jax.ShapeDtypeStruct((_ROWS, _N), jnp.float32),
        mesh=mesh,
        scratch_types=[
            pltpu.VMEM((_N,), jnp.float32),
            pltpu.VMEM((_N,), jnp.float32),
            pltpu.VMEM((_N,), jnp.float32),
            pltpu.VMEM((_N // _GW,), jnp.float32),
            pltpu.VMEM((_L * _PV,), jnp.int32),
            pltpu.VMEM((_L * _CAP2,), jnp.float32),
            pltpu.VMEM((_L * _CAP2,), jnp.int32),
            pltpu.SemaphoreType.DMA,
            pltpu.SemaphoreType.DMA,
        ],
        compiler_params=pltpu.CompilerParams(needs_layout_passes=False),
    )(input_tensor)
